# Initial kernel scaffold; baseline (speedup 1.0000x reference)
import jax
import jax.numpy as jnp
from jax.experimental import pallas as pl


def kernel(indexes, codebook):
    raise NotImplementedError("write your pallas kernel here")


# speedup vs baseline: 5.3682x; 5.3682x over previous
import functools

import jax
import jax.numpy as jnp
from jax import lax
from jax.experimental import pallas as pl
from jax.experimental.pallas import tpu as pltpu
from jax.experimental.pallas import tpu_sc as plsc

_V = 8192
_D = 64
_B = 1048576
_LW = 128
_NR = _B // _LW
_G = 8


def _make_gather():
    info = plsc.get_sparse_core_info()
    nw = info.num_cores * info.num_subcores
    rows_per_w = _NR // nw
    n_iter = rows_per_w // _G

    mesh = plsc.VectorSubcoreMesh(core_axis_name="c", subcore_axis_name="s")

    @functools.partial(
        pl.kernel,
        mesh=mesh,
        out_type=jax.ShapeDtypeStruct((_NR, _LW, _D), jnp.float32),
        scratch_types=[
            pltpu.VMEM((_G, _LW), jnp.int32),
            pltpu.VMEM((_G, _LW, _D), jnp.float32),
            pltpu.SemaphoreType.DMA,
        ],
        compiler_params=pltpu.CompilerParams(use_tc_tiling_on_sc=False),
    )
    def gather_kernel(idx_hbm, table_hbm, out_hbm, idx_v, rows_v, sem):
        wid = lax.axis_index("s") * info.num_cores + lax.axis_index("c")
        base = wid * rows_per_w

        def body(i, carry):
            r0 = base + i * _G
            pltpu.sync_copy(idx_hbm.at[pl.ds(r0, _G)], idx_v)
            copies = [
                pltpu.async_copy(table_hbm.at[idx_v.at[j]], rows_v.at[j], sem)
                for j in range(_G)
            ]
            for c in copies:
                c.wait()
            pltpu.sync_copy(rows_v, out_hbm.at[pl.ds(r0, _G)])
            return carry

        lax.fori_loop(0, n_iter, body, 0)

    return gather_kernel


_gather = _make_gather()


def kernel(indexes, codebook):
    idx2 = indexes.astype(jnp.int32).reshape(_NR, _LW)
    out = _gather(idx2, codebook)
    return out.reshape(_B, _D)


# speedup vs baseline: 5.4069x; 1.0072x over previous
import functools

import jax
import jax.numpy as jnp
from jax import lax
from jax.experimental import pallas as pl
from jax.experimental.pallas import tpu as pltpu
from jax.experimental.pallas import tpu_sc as plsc

_V = 8192
_D = 64
_B = 1048576
_LW = 128
_NR = _B // _LW
_G = 4
_NBUF = 2


def _make_gather():
    info = plsc.get_sparse_core_info()
    nw = info.num_cores * info.num_subcores
    rows_per_w = _NR // nw
    n_iter = rows_per_w // _G

    mesh = plsc.VectorSubcoreMesh(core_axis_name="c", subcore_axis_name="s")

    @functools.partial(
        pl.kernel,
        mesh=mesh,
        out_type=jax.ShapeDtypeStruct((_NR, _LW, _D), jnp.float32),
        scratch_types=[
            pltpu.VMEM((_NBUF, _G, _LW), jnp.int32),
            pltpu.VMEM((_NBUF, _G, _LW, _D), jnp.float32),
            [pltpu.SemaphoreType.DMA] * _NBUF,
            [pltpu.SemaphoreType.DMA] * _NBUF,
        ],
        compiler_params=pltpu.CompilerParams(use_tc_tiling_on_sc=False),
    )
    def gather_kernel(idx_hbm, table_hbm, out_hbm, idx_v, rows_v, g_sems, w_sems):
        wid = lax.axis_index("s") * info.num_cores + lax.axis_index("c")
        base = wid * rows_per_w

        def fill(g, b):
            r0 = base + g * _G
            pltpu.sync_copy(idx_hbm.at[pl.ds(r0, _G)], idx_v.at[b])
            for j in range(_G):
                pltpu.async_copy(
                    table_hbm.at[idx_v.at[b].at[j]], rows_v.at[b].at[j], g_sems[b]
                )

        def drain_gathers(b):
            for j in range(_G):
                pltpu.make_async_copy(
                    table_hbm.at[idx_v.at[b].at[j]], rows_v.at[b].at[j], g_sems[b]
                ).wait()

        def write(g, b):
            r0 = base + g * _G
            pltpu.async_copy(rows_v.at[b], out_hbm.at[pl.ds(r0, _G)], w_sems[b])

        def wait_write(g, b):
            r0 = base + g * _G
            pltpu.make_async_copy(rows_v.at[b], out_hbm.at[pl.ds(r0, _G)], w_sems[b]).wait()

        for b in range(_NBUF):
            fill(b, b)
        for b in range(_NBUF):
            drain_gathers(b)
            write(b, b)

        def body(p, carry):
            for b in range(_NBUF):
                g = _NBUF + p * _NBUF + b
                wait_write(g, b)
                fill(g, b)
                drain_gathers(b)
                write(g, b)
            return carry

        lax.fori_loop(0, (n_iter - _NBUF) // _NBUF, body, 0)

        for b in range(_NBUF):
            wait_write(0, b)

    return gather_kernel


_gather = _make_gather()


def kernel(indexes, codebook):
    idx2 = indexes.astype(jnp.int32).reshape(_NR, _LW)
    out = _gather(idx2, codebook)
    return out.reshape(_B, _D)


# speedup vs baseline: 5.9786x; 1.1057x over previous
import functools

import jax
import jax.numpy as jnp
from jax import lax
from jax.experimental import pallas as pl
from jax.experimental.pallas import tpu as pltpu
from jax.experimental.pallas import tpu_sc as plsc

_V = 8192
_D = 64
_B = 1048576
_LW = 128
_NR = _B // _LW
_G = 4
_NBUF = 2


def _make_gather():
    info = plsc.get_sparse_core_info()
    nw = info.num_cores * info.num_subcores
    rows_per_w = _NR // nw
    n_iter = rows_per_w // _G

    mesh = plsc.VectorSubcoreMesh(core_axis_name="c", subcore_axis_name="s")

    @functools.partial(
        pl.kernel,
        mesh=mesh,
        out_type=jax.ShapeDtypeStruct((_NR, _LW, _D), jnp.float32),
        scratch_types=[
            pltpu.VMEM_SHARED((_V, _D), jnp.float32),
            pltpu.VMEM((_NBUF, _G, _LW), jnp.int32),
            pltpu.VMEM((_NBUF, _G, _LW, _D), jnp.float32),
            [pltpu.SemaphoreType.DMA] * _NBUF,
            [pltpu.SemaphoreType.DMA] * _NBUF,
        ],
        compiler_params=pltpu.CompilerParams(use_tc_tiling_on_sc=False),
    )
    def gather_kernel(idx_hbm, table_hbm, out_hbm, table_sp, idx_v, rows_v,
                      g_sems, w_sems):
        sid = lax.axis_index("s")
        wid = sid * info.num_cores + lax.axis_index("c")
        base = wid * rows_per_w

        @pl.when(sid == 0)
        def _():
            pltpu.sync_copy(table_hbm, table_sp)

        plsc.subcore_barrier()

        def fill(g, b):
            r0 = base + g * _G
            pltpu.sync_copy(idx_hbm.at[pl.ds(r0, _G)], idx_v.at[b])
            for j in range(_G):
                pltpu.async_copy(
                    table_sp.at[idx_v.at[b].at[j]], rows_v.at[b].at[j], g_sems[b]
                )

        def drain_gathers(b):
            for j in range(_G):
                pltpu.make_async_copy(
                    table_sp.at[idx_v.at[b].at[j]], rows_v.at[b].at[j], g_sems[b]
                ).wait()

        def write(g, b):
            r0 = base + g * _G
            pltpu.async_copy(rows_v.at[b], out_hbm.at[pl.ds(r0, _G)], w_sems[b])

        def wait_write(g, b):
            r0 = base + g * _G
            pltpu.make_async_copy(rows_v.at[b], out_hbm.at[pl.ds(r0, _G)], w_sems[b]).wait()

        for b in range(_NBUF):
            fill(b, b)
        for b in range(_NBUF):
            drain_gathers(b)
            write(b, b)

        def body(p, carry):
            for b in range(_NBUF):
                g = _NBUF + p * _NBUF + b
                wait_write(g, b)
                fill(g, b)
                drain_gathers(b)
                write(g, b)
            return carry

        lax.fori_loop(0, (n_iter - _NBUF) // _NBUF, body, 0)

        for b in range(_NBUF):
            wait_write(0, b)

    return gather_kernel


_gather = _make_gather()


def kernel(indexes, codebook):
    idx2 = indexes.astype(jnp.int32).reshape(_NR, _LW)
    out = _gather(idx2, codebook)
    return out.reshape(_B, _D)


# speedup vs baseline: 7.9263x; 1.3258x over previous
import functools

import jax
import jax.numpy as jnp
from jax import lax
from jax.experimental import pallas as pl
from jax.experimental.pallas import tpu as pltpu
from jax.experimental.pallas import tpu_sc as plsc

_V = 8192
_D = 64
_B = 1048576
_C = 8192
_NCH = _B // _C
_CB = _C // 128
_NB = 2


def _make_gather():
    info = plsc.get_sparse_core_info()
    mesh = plsc.VectorSubcoreMesh(core_axis_name="c", subcore_axis_name="s")

    @functools.partial(
        pl.kernel,
        mesh=mesh,
        out_type=jax.ShapeDtypeStruct((8, _B // 128, 8, 128), jnp.float32),
        scratch_types=[
            pltpu.VMEM((2, _V), jnp.float32),
            pltpu.VMEM((_NB, _C), jnp.int32),
            pltpu.VMEM((_NB, 2, _CB, 128), jnp.float32),
            [pltpu.SemaphoreType.DMA] * _NB,
            [pltpu.SemaphoreType.DMA] * _NB,
        ],
        compiler_params=pltpu.CompilerParams(
            use_tc_tiling_on_sc=False, needs_layout_passes=False),
    )
    def gather_kernel(idx_hbm, ct_hbm, out_hbm, cols, idxb, outb, isems, wsems):
        wid = lax.axis_index("s") * info.num_cores + lax.axis_index("c")
        d0 = wid * 2
        a = d0 // 8
        c = d0 % 8

        pltpu.sync_copy(ct_hbm.at[pl.ds(d0, 2)], cols)

        def start_idx(q, b):
            off = pl.multiple_of(q * _C, 8)
            pltpu.async_copy(idx_hbm.at[pl.ds(off, _C)], idxb.at[b], isems[b])

        def wait_idx(q, b):
            off = pl.multiple_of(q * _C, 8)
            pltpu.make_async_copy(
                idx_hbm.at[pl.ds(off, _C)], idxb.at[b], isems[b]).wait()

        def compute(b):
            def bb_body(bb, carry):
                for u in range(8):
                    ivec = idxb[b, pl.ds(bb * 128 + u * 16, 16)]
                    for dl in range(2):
                        vals = plsc.load_gather(cols.at[dl], [ivec])
                        outb[b, dl, bb, pl.ds(u * 16, 16)] = vals
                return carry
            lax.fori_loop(0, _CB, bb_body, 0)

        def start_write(q, b):
            for dl in range(2):
                pltpu.async_copy(
                    outb.at[b, dl], out_hbm.at[a, pl.ds(q * _CB, _CB), c + dl],
                    wsems[b])

        def wait_write(q, b):
            for dl in range(2):
                pltpu.make_async_copy(
                    outb.at[b, dl], out_hbm.at[a, pl.ds(q * _CB, _CB), c + dl],
                    wsems[b]).wait()

        def body(p, carry):
            for bpar in range(_NB):
                pp = p * _NB + bpar
                q = pp - 1

                @pl.when(pp < _NCH)
                def _():
                    start_idx(pp, bpar)

                @pl.when(jnp.logical_and(q >= 0, q < _NCH))
                def _():
                    b1 = bpar ^ 1

                    @pl.when(q >= 2)
                    def _():
                        wait_write(q - 2, b1)

                    wait_idx(q, b1)
                    compute(b1)
                    start_write(q, b1)
            return carry

        lax.fori_loop(0, (_NCH + 2) // _NB, body, 0)

        for q in (_NCH - 2, _NCH - 1):
            wait_write(q, q % _NB)

    return gather_kernel


_gather = _make_gather()


def kernel(indexes, codebook):
    idx = indexes.astype(jnp.int32)
    ct = codebook.T
    out4 = _gather(idx, ct)
    return out4.transpose(1, 3, 0, 2).reshape(_B, _D)


# speedup vs baseline: 19.3706x; 2.4438x over previous
import functools

import jax
import jax.numpy as jnp
from jax import lax
from jax.experimental import pallas as pl
from jax.experimental.pallas import tpu as pltpu
from jax.experimental.pallas import tpu_sc as plsc

_V = 8192
_D = 64
_B = 1048576
_C = 8192
_NCH = _B // _C
_CB = _C // 128
_NB = 2


def _make_gather():
    info = plsc.get_sparse_core_info()
    mesh = plsc.VectorSubcoreMesh(core_axis_name="c", subcore_axis_name="s")

    @functools.partial(
        pl.kernel,
        mesh=mesh,
        out_type=jax.ShapeDtypeStruct((8, _B // 128, 8, 128), jnp.float32),
        scratch_types=[
            pltpu.VMEM((2, _V), jnp.float32),
            pltpu.VMEM((_NB, _C), jnp.int32),
            pltpu.VMEM((_NB, 2, _CB, 128), jnp.float32),
            [pltpu.SemaphoreType.DMA] * _NB,
            [pltpu.SemaphoreType.DMA] * _NB,
        ],
        compiler_params=pltpu.CompilerParams(
            use_tc_tiling_on_sc=False, needs_layout_passes=False),
    )
    def gather_kernel(idx_hbm, ct_hbm, out_hbm, cols, idxb, outb, isems, wsems):
        wid = lax.axis_index("s") * info.num_cores + lax.axis_index("c")
        d0 = wid * 2
        a = d0 // 8
        c = d0 % 8

        pltpu.sync_copy(ct_hbm.at[pl.ds(d0, 2)], cols)

        def start_idx(q, b):
            off = pl.multiple_of(q * _C, 8)
            pltpu.async_copy(idx_hbm.at[pl.ds(off, _C)], idxb.at[b], isems[b])

        def wait_idx(q, b):
            off = pl.multiple_of(q * _C, 8)
            pltpu.make_async_copy(
                idx_hbm.at[pl.ds(off, _C)], idxb.at[b], isems[b]).wait()

        def compute(b):
            @plsc.parallel_loop(0, _CB, unroll=4)
            def bb_body(bb):
                for u in range(8):
                    ivec = idxb[b, pl.ds(bb * 128 + u * 16, 16)]
                    for dl in range(2):
                        vals = plsc.load_gather(cols.at[dl], [ivec])
                        outb[b, dl, bb, pl.ds(u * 16, 16)] = vals

        def start_write(q, b):
            for dl in range(2):
                pltpu.async_copy(
                    outb.at[b, dl], out_hbm.at[a, pl.ds(q * _CB, _CB), c + dl],
                    wsems[b])

        def wait_write(q, b):
            for dl in range(2):
                pltpu.make_async_copy(
                    outb.at[b, dl], out_hbm.at[a, pl.ds(q * _CB, _CB), c + dl],
                    wsems[b]).wait()

        def body(p, carry):
            for bpar in range(_NB):
                pp = p * _NB + bpar
                q = pp - 1

                @pl.when(pp < _NCH)
                def _():
                    start_idx(pp, bpar)

                @pl.when(jnp.logical_and(q >= 0, q < _NCH))
                def _():
                    b1 = bpar ^ 1

                    @pl.when(q >= 2)
                    def _():
                        wait_write(q - 2, b1)

                    wait_idx(q, b1)
                    compute(b1)
                    start_write(q, b1)
            return carry

        lax.fori_loop(0, (_NCH + 2) // _NB, body, 0)

        for q in (_NCH - 2, _NCH - 1):
            wait_write(q, q % _NB)

    return gather_kernel


_gather = _make_gather()


def kernel(indexes, codebook):
    idx = indexes.astype(jnp.int32)
    ct = codebook.T
    out4 = _gather(idx, ct)
    return out4.transpose(1, 3, 0, 2).reshape(_B, _D)


# speedup vs baseline: 20.1252x; 1.0390x over previous
import functools

import jax
import jax.numpy as jnp
from jax import lax
from jax.experimental import pallas as pl
from jax.experimental.pallas import tpu as pltpu
from jax.experimental.pallas import tpu_sc as plsc

_V = 8192
_D = 64
_B = 1048576
_C = 16384
_NCH = _B // _C
_CB = _C // 128
_NB = 2


def _make_gather():
    info = plsc.get_sparse_core_info()
    mesh = plsc.VectorSubcoreMesh(core_axis_name="c", subcore_axis_name="s")

    @functools.partial(
        pl.kernel,
        mesh=mesh,
        out_type=jax.ShapeDtypeStruct((8, _B // 128, 8, 128), jnp.float32),
        scratch_types=[
            pltpu.VMEM((2, _V), jnp.float32),
            pltpu.VMEM((_NB, _C), jnp.int32),
            pltpu.VMEM((_NB, 2, _CB, 128), jnp.float32),
            [pltpu.SemaphoreType.DMA] * _NB,
            [pltpu.SemaphoreType.DMA] * _NB,
        ],
        compiler_params=pltpu.CompilerParams(
            use_tc_tiling_on_sc=False, needs_layout_passes=False),
    )
    def gather_kernel(idx_hbm, ct_hbm, out_hbm, cols, idxb, outb, isems, wsems):
        wid = lax.axis_index("s") * info.num_cores + lax.axis_index("c")
        d0 = wid * 2
        a = d0 // 8
        c = d0 % 8

        pltpu.sync_copy(ct_hbm.at[pl.ds(d0, 2)], cols)

        def start_idx(q, b):
            off = pl.multiple_of(q * _C, 8)
            pltpu.async_copy(idx_hbm.at[pl.ds(off, _C)], idxb.at[b], isems[b])

        def wait_idx(q, b):
            off = pl.multiple_of(q * _C, 8)
            pltpu.make_async_copy(
                idx_hbm.at[pl.ds(off, _C)], idxb.at[b], isems[b]).wait()

        def compute(b):
            @plsc.parallel_loop(0, _CB, unroll=8)
            def bb_body(bb):
                for u in range(8):
                    ivec = idxb[b, pl.ds(bb * 128 + u * 16, 16)]
                    for dl in range(2):
                        vals = plsc.load_gather(cols.at[dl], [ivec])
                        outb[b, dl, bb, pl.ds(u * 16, 16)] = vals

        def start_write(q, b):
            for dl in range(2):
                pltpu.async_copy(
                    outb.at[b, dl], out_hbm.at[a, pl.ds(q * _CB, _CB), c + dl],
                    wsems[b])

        def wait_write(q, b):
            for dl in range(2):
                pltpu.make_async_copy(
                    outb.at[b, dl], out_hbm.at[a, pl.ds(q * _CB, _CB), c + dl],
                    wsems[b]).wait()

        def body(p, carry):
            for bpar in range(_NB):
                pp = p * _NB + bpar
                q = pp - 1

                @pl.when(pp < _NCH)
                def _():
                    start_idx(pp, bpar)

                @pl.when(jnp.logical_and(q >= 0, q < _NCH))
                def _():
                    b1 = bpar ^ 1

                    @pl.when(q >= 2)
                    def _():
                        wait_write(q - 2, b1)

                    wait_idx(q, b1)
                    compute(b1)
                    start_write(q, b1)
            return carry

        lax.fori_loop(0, (_NCH + 2) // _NB, body, 0)

        for q in (_NCH - 2, _NCH - 1):
            wait_write(q, q % _NB)

    return gather_kernel


_gather = _make_gather()


def kernel(indexes, codebook):
    idx = indexes.astype(jnp.int32)
    ct = codebook.T
    out4 = _gather(idx, ct)
    return out4.transpose(1, 3, 0, 2).reshape(_B, _D)
